# baseline (speedup 1.0000x reference)
import functools

import jax
import jax.numpy as jnp
from jax import lax
from jax.experimental import pallas as pl
from jax.experimental.pallas import tpu as pltpu
from jax.experimental.pallas import tpu_sc as plsc

_D = 32
_B = 16384
_S = 50
_TD = _D // 8
_TB = _B // 128
_NC = 2
_NS = 16
_NW = _NC * _NS
_BPW = _B // _NW
_TBW = _TB // _NW
_CPW = _S * _TBW
_NBUF = 4
_LA = 2

_mesh = plsc.VectorSubcoreMesh(core_axis_name="c", subcore_axis_name="s")


@functools.partial(
    pl.kernel,
    out_type=jax.ShapeDtypeStruct((_S, _TD, _TB, 8, 128), jnp.float32),
    mesh=_mesh,
    scratch_types=[
        pltpu.VMEM((_BPW, _S), jnp.int32),
        pltpu.VMEM((_S, _BPW), jnp.int32),
        pltpu.VMEM((_NBUF, 128, _D), jnp.float32),
        pltpu.VMEM((_NBUF, 8, 541), jnp.float32),
        pltpu.SemaphoreType.DMA((_NBUF,)),
        pltpu.SemaphoreType.DMA((_NBUF,)),
    ],
    compiler_params=pltpu.CompilerParams(
        use_tc_tiling_on_sc=False, needs_layout_passes=False
    ),
)
def _lookup(data_hbm, idx_hbm, out_hbm, idx_v, idx_t, rows_v, tbuf, gsem, osem):
    wid = lax.axis_index("s") * _NC + lax.axis_index("c")
    base_b = wid * _BPW
    pltpu.sync_copy(idx_hbm.at[pl.ds(base_b, _BPW)], idx_v)

    lanes = lax.iota(jnp.int32, 16)
    dr_v = lax.bitwise_and(lanes, 7)
    td_base = lax.shift_right_logical(lanes, 3)
    colb = [(td_base + 2 * h) * 136 for h in range(2)]
    def idx_t_body(bg, _):
        rows_ids = lanes + bg * 16
        for s in range(_S):
            v = plsc.load_gather(idx_v, [rows_ids, jnp.full((16,), s, jnp.int32)])
            idx_t[s, pl.ds(bg * 16, 16)] = v
        return ()

    lax.fori_loop(0, _BPW // 16, idx_t_body, (), unroll=False)

    def offsets(j):
        s = j // _TBW
        tb = j % _TBW
        return idx_t.at[s].at[pl.ds(tb * 128, 128)]

    def chunk_writes(j, b):
        s = j // _TBW
        tb = wid * _TBW + (j % _TBW)
        return [
            pltpu.make_async_copy(
                tbuf.at[b].at[:, pl.ds(td * 136, 128)],
                out_hbm.at[s].at[td].at[tb],
                osem.at[b],
            )
            for td in range(_TD)
        ]

    for b in range(_LA):
        pltpu.async_copy(data_hbm.at[offsets(b)], rows_v.at[b], gsem.at[b])

    def group(g, _):
        for b in range(_NBUF):
            j = g * _NBUF + b
            pltpu.make_async_copy(
                data_hbm.at[offsets(j)], rows_v.at[b], gsem.at[b]
            ).wait()

            @pl.when(j >= _NBUF)
            def _():
                for cp in chunk_writes(j - _NBUF, b):
                    cp.wait()

            def tr_body(bc, _):
                for h in range(2):
                    v = rows_v[b, bc, pl.ds(h * 16, 16)]
                    plsc.store_scatter(
                        tbuf.at[b], [dr_v, colb[h] + bc], v
                    )
                return ()

            lax.fori_loop(0, 128, tr_body, (), unroll=8)

            plsc.subcore_barrier()
            for cp in chunk_writes(j, b):
                cp.start()

            nj = j + _LA

            @pl.when(nj < _CPW)
            def _():
                pltpu.async_copy(
                    data_hbm.at[offsets(nj)],
                    rows_v.at[(b + _LA) % _NBUF],
                    gsem.at[(b + _LA) % _NBUF],
                )
        return ()

    lax.fori_loop(0, _CPW // _NBUF, group, (), unroll=False)
    for k in range(_CPW - _NBUF, _CPW):
        for cp in chunk_writes(k, k % _NBUF):
            cp.wait()


_V = 1000000
_BW = 6400


def _tc_transpose_body(x_ref, o_ref):
    o_ref[...] = x_ref[...].T


_tc_transpose = pl.pallas_call(
    _tc_transpose_body,
    grid=(pl.cdiv(_V, _BW),),
    in_specs=[pl.BlockSpec((_D, _BW), lambda i: (0, i))],
    out_specs=pl.BlockSpec((_BW, _D), lambda i: (i, 0)),
    out_shape=jax.ShapeDtypeStruct((_V, _D), jnp.float32),
)


def kernel(data, input_ids):
    data_lin = _tc_transpose(jnp.swapaxes(data, 0, 1))
    y = _lookup(data_lin, input_ids.astype(jnp.int32))
    z = y.transpose(0, 1, 3, 2, 4).reshape(_S, _D, _B)
    return z.transpose(2, 0, 1)


# speedup vs baseline: 1.1622x; 1.1622x over previous
import functools

import jax
import jax.numpy as jnp
from jax import lax
from jax.experimental import pallas as pl
from jax.experimental.pallas import tpu as pltpu
from jax.experimental.pallas import tpu_sc as plsc

_D = 32
_B = 16384
_S = 50
_TD = _D // 8
_TB = _B // 128
_NC = 2
_NS = 16
_NW = _NC * _NS
_BPW = _B // _NW
_TBW = _TB // _NW
_CPW = _S * _TBW
_NBUF = 5
_LA = 3

_mesh = plsc.VectorSubcoreMesh(core_axis_name="c", subcore_axis_name="s")


@functools.partial(
    pl.kernel,
    out_type=jax.ShapeDtypeStruct((_S, _TD, _TB, 8, 128), jnp.float32),
    mesh=_mesh,
    scratch_types=[
        pltpu.VMEM((_BPW, _S), jnp.int32),
        pltpu.VMEM((_S, _BPW), jnp.int32),
        pltpu.VMEM((_NBUF, 128, _D), jnp.float32),
        pltpu.VMEM((_NBUF, 8, 541), jnp.float32),
        pltpu.SemaphoreType.DMA((_NBUF,)),
        pltpu.SemaphoreType.DMA((_NBUF,)),
    ],
    compiler_params=pltpu.CompilerParams(
        use_tc_tiling_on_sc=False, needs_layout_passes=False
    ),
)
def _lookup(data_hbm, idx_hbm, out_hbm, idx_v, idx_t, rows_v, tbuf, gsem, osem):
    wid = lax.axis_index("s") * _NC + lax.axis_index("c")
    base_b = wid * _BPW
    pltpu.sync_copy(idx_hbm.at[pl.ds(base_b, _BPW)], idx_v)

    lanes = lax.iota(jnp.int32, 16)
    dr_v = lax.bitwise_and(lanes, 7)
    td_base = lax.shift_right_logical(lanes, 3)
    colb = [(td_base + 2 * h) * 136 for h in range(2)]
    def idx_t_body(bg, _):
        rows_ids = lanes + bg * 16
        for s in range(_S):
            v = plsc.load_gather(idx_v, [rows_ids, jnp.full((16,), s, jnp.int32)])
            idx_t[s, pl.ds(bg * 16, 16)] = v
        return ()

    lax.fori_loop(0, _BPW // 16, idx_t_body, (), unroll=False)

    def offsets(j):
        s = j // _TBW
        tb = j % _TBW
        return idx_t.at[s].at[pl.ds(tb * 128, 128)]

    def chunk_writes(j, b):
        s = j // _TBW
        tb = wid * _TBW + (j % _TBW)
        return [
            pltpu.make_async_copy(
                tbuf.at[b].at[:, pl.ds(td * 136, 128)],
                out_hbm.at[s].at[td].at[tb],
                osem.at[b],
            )
            for td in range(_TD)
        ]

    for b in range(_LA):
        pltpu.async_copy(data_hbm.at[offsets(b)], rows_v.at[b], gsem.at[b])

    def group(g, _):
        for b in range(_NBUF):
            j = g * _NBUF + b
            pltpu.make_async_copy(
                data_hbm.at[offsets(j)], rows_v.at[b], gsem.at[b]
            ).wait()

            @pl.when(j >= _NBUF)
            def _():
                for cp in chunk_writes(j - _NBUF, b):
                    cp.wait()

            def tr_body(bc, _):
                for h in range(2):
                    v = rows_v[b, bc, pl.ds(h * 16, 16)]
                    plsc.store_scatter(
                        tbuf.at[b], [dr_v, colb[h] + bc], v
                    )
                return ()

            lax.fori_loop(0, 128, tr_body, (), unroll=16)

            plsc.subcore_barrier()
            for cp in chunk_writes(j, b):
                cp.start()

            nj = j + _LA

            @pl.when(nj < _CPW)
            def _():
                pltpu.async_copy(
                    data_hbm.at[offsets(nj)],
                    rows_v.at[(b + _LA) % _NBUF],
                    gsem.at[(b + _LA) % _NBUF],
                )
        return ()

    lax.fori_loop(0, _CPW // _NBUF, group, (), unroll=False)
    for k in range(_CPW - _NBUF, _CPW):
        for cp in chunk_writes(k, k % _NBUF):
            cp.wait()


def kernel(data, input_ids):
    y = _lookup(data, input_ids.astype(jnp.int32))
    z = y.transpose(0, 1, 3, 2, 4).reshape(_S, _D, _B)
    return z.transpose(2, 0, 1)


# speedup vs baseline: 1.1625x; 1.0003x over previous
import functools

import jax
import jax.numpy as jnp
from jax import lax
from jax.experimental import pallas as pl
from jax.experimental.pallas import tpu as pltpu
from jax.experimental.pallas import tpu_sc as plsc

_D = 32
_B = 16384
_S = 50
_TD = _D // 8
_TB = _B // 128
_NC = 2
_NS = 16
_NW = _NC * _NS
_BPW = _B // _NW
_TBW = _TB // _NW
_CPW = _S * _TBW
_NBUF = 5
_LA = 4

_mesh = plsc.VectorSubcoreMesh(core_axis_name="c", subcore_axis_name="s")


@functools.partial(
    pl.kernel,
    out_type=jax.ShapeDtypeStruct((_S, _TD, _TB, 8, 128), jnp.float32),
    mesh=_mesh,
    scratch_types=[
        pltpu.VMEM((_BPW, _S), jnp.int32),
        pltpu.VMEM((_S, _BPW), jnp.int32),
        pltpu.VMEM((_NBUF, 128, _D), jnp.float32),
        pltpu.VMEM((_NBUF, 8, 541), jnp.float32),
        pltpu.SemaphoreType.DMA((_NBUF,)),
        pltpu.SemaphoreType.DMA((_NBUF,)),
    ],
    compiler_params=pltpu.CompilerParams(
        use_tc_tiling_on_sc=False, needs_layout_passes=False
    ),
)
def _lookup(data_hbm, idx_hbm, out_hbm, idx_v, idx_t, rows_v, tbuf, gsem, osem):
    wid = lax.axis_index("s") * _NC + lax.axis_index("c")
    base_b = wid * _BPW
    pltpu.sync_copy(idx_hbm.at[pl.ds(base_b, _BPW)], idx_v)

    lanes = lax.iota(jnp.int32, 16)
    dr_v = lax.bitwise_and(lanes, 7)
    td_base = lax.shift_right_logical(lanes, 3)
    colb = [(td_base + 2 * h) * 136 for h in range(2)]
    def idx_t_body(bg, _):
        rows_ids = lanes + bg * 16
        for s in range(_S):
            v = plsc.load_gather(idx_v, [rows_ids, jnp.full((16,), s, jnp.int32)])
            idx_t[s, pl.ds(bg * 16, 16)] = v
        return ()

    lax.fori_loop(0, _BPW // 16, idx_t_body, (), unroll=False)

    def offsets(j):
        s = j // _TBW
        tb = j % _TBW
        return idx_t.at[s].at[pl.ds(tb * 128, 128)]

    def chunk_writes(j, b):
        s = j // _TBW
        tb = wid * _TBW + (j % _TBW)
        return [
            pltpu.make_async_copy(
                tbuf.at[b].at[:, pl.ds(td * 136, 128)],
                out_hbm.at[s].at[td].at[tb],
                osem.at[b],
            )
            for td in range(_TD)
        ]

    for b in range(_LA):
        pltpu.async_copy(data_hbm.at[offsets(b)], rows_v.at[b], gsem.at[b])

    def group(g, _):
        for b in range(_NBUF):
            j = g * _NBUF + b
            pltpu.make_async_copy(
                data_hbm.at[offsets(j)], rows_v.at[b], gsem.at[b]
            ).wait()

            @pl.when(j >= _NBUF)
            def _():
                for cp in chunk_writes(j - _NBUF, b):
                    cp.wait()

            def tr_body(bc, _):
                for h in range(2):
                    v = rows_v[b, bc, pl.ds(h * 16, 16)]
                    plsc.store_scatter(
                        tbuf.at[b], [dr_v, colb[h] + bc], v
                    )
                return ()

            lax.fori_loop(0, 128, tr_body, (), unroll=16)

            plsc.subcore_barrier()
            for cp in chunk_writes(j, b):
                cp.start()

            nj = j + _LA

            @pl.when(nj < _CPW)
            def _():
                pltpu.async_copy(
                    data_hbm.at[offsets(nj)],
                    rows_v.at[(b + _LA) % _NBUF],
                    gsem.at[(b + _LA) % _NBUF],
                )
        return ()

    lax.fori_loop(0, _CPW // _NBUF, group, (), unroll=False)
    for k in range(_CPW - _NBUF, _CPW):
        for cp in chunk_writes(k, k % _NBUF):
            cp.wait()


def kernel(data, input_ids):
    y = _lookup(data, input_ids.astype(jnp.int32))
    z = y.transpose(0, 1, 3, 2, 4).reshape(_S, _D, _B)
    return z.transpose(2, 0, 1)
